# baseline (speedup 1.0000x reference)
import functools
import numpy as np
import jax
import jax.numpy as jnp
from jax.experimental import pallas as pl
from jax.experimental.pallas import tpu as pltpu

B = 1024
A = 2
HW = 256
C = 26
NAGENTS = B * A
ROW = HW * C
R = 256
GRID = NAGENTS // R
BIG = 4096

_lane = np.arange(ROW)
_CELLS = jnp.array((_lane // C)[None, :], dtype=jnp.int32)
_KEYSRC = jnp.array(np.where(_lane % C == 0, _lane // C, BIG)[None, :],
                    dtype=jnp.int32)


def _fold(x, op):
    n = x.shape[1]
    while n > C:
        n //= 2
        x = op(x[:, :n], x[:, n:2 * n])
    return x


def _body(obs_ref, rew_ref, cells_ref, keysrc_ref, out_ref):
    blk = obs_ref[...]
    cells = cells_ref[...]
    keysrc = keysrc_ref[...]

    sums = _fold(blk, jnp.add)
    maxs = _fold(blk, jnp.maximum)
    key26 = _fold(jnp.where(blk > 0, keysrc, BIG), jnp.minimum)
    key = key26[:, 0]

    found = key < BIG
    ax = key >> 4
    ay = key & 15
    interior = found & (ax >= 1) & (ax <= 14) & (ay >= 1) & (ay <= 14)
    loc = jnp.where(interior, (ax - 1) * 14 + (ay - 1), -1)

    s2, s3, s4, s5 = sums[:, 2], sums[:, 3], sums[:, 4], sums[:, 5]
    d = jnp.zeros((R,), jnp.int32)
    best = s2
    d = jnp.where(s3 > best, 1, d)
    best = jnp.maximum(best, s3)
    d = jnp.where(s4 > best, 2, d)
    best = jnp.maximum(best, s4)
    d = jnp.where(s5 > best, 3, d)
    dr = jnp.where(d == 0, -1, jnp.where(d == 1, 1, 0))
    dc = jnp.where(d == 2, 1, jnp.where(d == 3, -1, 0))
    axr = jnp.where(found, ax, -1)
    ayr = jnp.where(found, ay, -1)
    fx = axr + dr
    fy = ayr + dc
    fvalid = (fx >= 0) & (fx < 16) & (fy >= 0) & (fy < 16)
    facing = jnp.where(fvalid, fx * 16 + fy, -1)

    p = jnp.where(found, key, 255)
    pv = _fold(jnp.where(cells == p[:, None], blk, -3.4e38), jnp.maximum)
    pot = pv[:, 10] > 0
    soup = pv[:, 21] > 0
    plate = pv[:, 22] > 0
    onion = pv[:, 23] > 0
    carrying = jnp.where(onion, 1, jnp.where(soup & (~pot), 3,
               jnp.where(plate, 2, 0)))

    s16 = sums[:, 16]
    m20 = maxs[:, 20]
    m21 = maxs[:, 21]
    pot_state = jnp.where(m21 > 0., 10,
        jnp.where(m20 > 0.,
            jnp.where(m20 >= 17., 4, jnp.where(m20 >= 13., 5, jnp.where(m20 >= 9., 6,
            jnp.where(m20 >= 5., 7, jnp.where(m20 >= 2., 8, 9))))),
            jnp.where(s16 == 0., 0, jnp.where(s16 == 1., 1,
            jnp.where(s16 == 2., 2, 3)))))

    rew = rew_ref[...]
    goal = (rew[:, 0] >= 20.0) | (rew[:, 1] >= 20.0)

    out_ref[...] = jnp.stack([
        loc.astype(jnp.float32),
        facing.astype(jnp.float32),
        carrying.astype(jnp.float32),
        pot_state.astype(jnp.float32),
        goal.astype(jnp.float32),
    ], axis=1)


@functools.partial(jax.jit, static_argnames=("interpret",))
def _run(obs2, rew_pairs, interpret=False):
    return pl.pallas_call(
        _body,
        grid=(GRID,),
        in_specs=[
            pl.BlockSpec((R, ROW), lambda i: (i, 0)),
            pl.BlockSpec((R, A), lambda i: (i, 0)),
            pl.BlockSpec((1, ROW), lambda i: (0, 0)),
            pl.BlockSpec((1, ROW), lambda i: (0, 0)),
        ],
        out_specs=pl.BlockSpec((R, 5), lambda i: (i, 0)),
        out_shape=jax.ShapeDtypeStruct((NAGENTS, 5), jnp.float32),
        compiler_params=pltpu.CompilerParams(
            dimension_semantics=("arbitrary",)),
        interpret=interpret,
    )(obs2, rew_pairs, _CELLS, _KEYSRC)


def kernel(obs, rewards):
    obs2 = obs.reshape(NAGENTS, ROW)
    rew_pairs = jnp.broadcast_to(
        rewards.reshape(B, 1, A), (B, A, A)).reshape(NAGENTS, A)
    out = _run(obs2, rew_pairs)
    return out.reshape(B, A, 5)
